# Initial kernel scaffold; baseline (speedup 1.0000x reference)
import jax
import jax.numpy as jnp
from jax.experimental import pallas as pl


def kernel(arg, weight):
    raise NotImplementedError("write your pallas kernel here")


# speedup vs baseline: 5.6431x; 5.6431x over previous
import functools

import jax
import jax.numpy as jnp
from jax import lax
from jax.experimental import pallas as pl
from jax.experimental.pallas import tpu as pltpu
from jax.experimental.pallas import tpu_sc as plsc

_B = 16384
_S = 200
_E = 3
_N = _B * _S
_NW = 32
_PER_W = _N // _NW
_BLK = 12800
_NBLK = _PER_W // _BLK


def _emb_kernel(idx_hbm, w_hbm, out_hbm, w_v, idx_v, out_v):
    wid = lax.axis_index("s") * 2 + lax.axis_index("c")
    pltpu.sync_copy(w_hbm, w_v)
    base = wid * _PER_W
    lane = lax.iota(jnp.int32, 16)

    def block_body(b, carry):
        off = base + b * _BLK
        pltpu.sync_copy(idx_hbm.at[pl.ds(off, _BLK)], idx_v)

        def vec_body(i, carry2):
            v = idx_v[pl.ds(i * 16, 16)]
            b3 = v * 3
            p0 = i * 48 + lane * 3
            w0 = plsc.load_gather(w_v, [b3])
            w1 = plsc.load_gather(w_v, [b3 + 1])
            w2 = plsc.load_gather(w_v, [b3 + 2])
            plsc.store_scatter(out_v, [p0], w0)
            plsc.store_scatter(out_v, [p0 + 1], w1)
            plsc.store_scatter(out_v, [p0 + 2], w2)
            return carry2

        lax.fori_loop(0, _BLK // 16, vec_body, 0)
        pltpu.sync_copy(out_v, out_hbm.at[pl.ds(off * 3, _BLK * 3)])
        return carry

    lax.fori_loop(0, _NBLK, block_body, 0)


@jax.jit
def _emb(idx_flat, wflat):
    mesh = plsc.VectorSubcoreMesh(core_axis_name="c", subcore_axis_name="s")
    run = functools.partial(
        pl.kernel,
        mesh=mesh,
        out_type=jax.ShapeDtypeStruct((_N * _E,), jnp.float32),
        scratch_types=[
            pltpu.VMEM((32,), jnp.float32),
            pltpu.VMEM((_BLK,), jnp.int32),
            pltpu.VMEM((_BLK * _E,), jnp.float32),
        ],
        compiler_params=pltpu.CompilerParams(needs_layout_passes=False),
    )(_emb_kernel)
    return run(idx_flat, wflat)


def kernel(arg, weight):
    idx_flat = arg.reshape(_N).astype(jnp.int32)
    wflat = jnp.pad(weight.reshape(_E * 10), (0, 2))
    out = _emb(idx_flat, wflat)
    return out.reshape(_B, _S, _E)


# speedup vs baseline: 7.4139x; 1.3138x over previous
import functools

import jax
import jax.numpy as jnp
from jax import lax
from jax.experimental import pallas as pl
from jax.experimental.pallas import tpu as pltpu
from jax.experimental.pallas import tpu_sc as plsc

_B = 16384
_S = 200
_E = 3
_NW = 32
_ROWS_W = _B // _NW
_ROWS_BLK = 64
_NBLK = _ROWS_W // _ROWS_BLK
_VECS = _ROWS_BLK * _S // 16


def _emb_kernel(idx_hbm, w_hbm, out_hbm, w_v, idx_v, out_v):
    wid = lax.axis_index("s") * 2 + lax.axis_index("c")
    pltpu.sync_copy(w_hbm, w_v)
    row0 = wid * _ROWS_W
    lane = lax.iota(jnp.int32, 16)
    d0 = jnp.full((16,), 0, jnp.int32)
    d1 = jnp.full((16,), 1, jnp.int32)
    d2 = jnp.full((16,), 2, jnp.int32)

    def block_body(b, carry):
        rb = row0 + b * _ROWS_BLK
        pltpu.sync_copy(idx_hbm.at[pl.ds(rb, _ROWS_BLK), :], idx_v)

        def vec_body(i, rc):
            r, c = rc
            v = plsc.load_gather(idx_v, [r, c])
            b3 = v * 3
            w0 = plsc.load_gather(w_v, [b3])
            w1 = plsc.load_gather(w_v, [b3 + 1])
            w2 = plsc.load_gather(w_v, [b3 + 2])
            plsc.store_scatter(out_v, [r, c, d0], w0)
            plsc.store_scatter(out_v, [r, c, d1], w1)
            plsc.store_scatter(out_v, [r, c, d2], w2)
            c2 = c + 16
            wrap = c2 >= _S
            return (jnp.where(wrap, r + 1, r), jnp.where(wrap, c2 - _S, c2))

        lax.fori_loop(0, _VECS, vec_body, (jnp.zeros((16,), jnp.int32), lane))
        pltpu.sync_copy(out_v, out_hbm.at[pl.ds(rb, _ROWS_BLK), :, :])
        return carry

    lax.fori_loop(0, _NBLK, block_body, 0)


@jax.jit
def _emb(idx, wflat):
    mesh = plsc.VectorSubcoreMesh(core_axis_name="c", subcore_axis_name="s")
    run = functools.partial(
        pl.kernel,
        mesh=mesh,
        out_type=jax.ShapeDtypeStruct((_B, _S, _E), jnp.float32),
        scratch_types=[
            pltpu.VMEM((32,), jnp.float32),
            pltpu.VMEM((_ROWS_BLK, _S), jnp.int32),
            pltpu.VMEM((_ROWS_BLK, _S, _E), jnp.float32),
        ],
        compiler_params=pltpu.CompilerParams(
            needs_layout_passes=False, use_tc_tiling_on_sc=False
        ),
    )(_emb_kernel)
    return run(idx, wflat)


def kernel(arg, weight):
    wflat = jnp.pad(weight.reshape(_E * 10), (0, 2))
    return _emb(arg.astype(jnp.int32), wflat)


# speedup vs baseline: 7.7785x; 1.0492x over previous
import functools

import jax
import jax.numpy as jnp
from jax import lax
from jax.experimental import pallas as pl
from jax.experimental.pallas import tpu as pltpu
from jax.experimental.pallas import tpu_sc as plsc

_B = 16384
_S = 200
_E = 3
_NW = 32
_ROWS_W = _B // _NW
_IDX_BLK = 8
_OUT_BLK = 4
_VECS = _OUT_BLK * _S // 16


def _emb_kernel(idx_hbm, w_hbm, out_hbm, w_v, idx_v, out_v):
    wid = lax.axis_index("s") * 2 + lax.axis_index("c")
    pltpu.sync_copy(w_hbm, w_v)
    row0 = wid * _ROWS_W
    lane = lax.iota(jnp.int32, 16)
    d0 = jnp.full((16,), 0, jnp.int32)
    d1 = jnp.full((16,), 1, jnp.int32)
    d2 = jnp.full((16,), 2, jnp.int32)

    def block_body(b, carry):
        rb = row0 + b * _IDX_BLK
        pltpu.sync_copy(idx_hbm.at[pl.ds(rb, _IDX_BLK), :], idx_v)

        for half in range(_IDX_BLK // _OUT_BLK):
            off = half * _OUT_BLK

            def vec_body(i, rc, off=off):
                r, c = rc
                v = plsc.load_gather(idx_v, [r + off, c])
                b3 = v * 3
                w0 = plsc.load_gather(w_v, [b3])
                w1 = plsc.load_gather(w_v, [b3 + 1])
                w2 = plsc.load_gather(w_v, [b3 + 2])
                plsc.store_scatter(out_v, [r, c, d0], w0)
                plsc.store_scatter(out_v, [r, c, d1], w1)
                plsc.store_scatter(out_v, [r, c, d2], w2)
                c2 = c + 16
                wrap = c2 >= _S
                return (
                    jnp.where(wrap, r + 1, r),
                    jnp.where(wrap, c2 - _S, c2),
                )

            lax.fori_loop(
                0, _VECS, vec_body, (jnp.zeros((16,), jnp.int32), lane)
            )
            pltpu.sync_copy(
                out_v, out_hbm.at[pl.ds(rb + off, _OUT_BLK), :, :]
            )
        return carry

    lax.fori_loop(0, _ROWS_W // _IDX_BLK, block_body, 0)


@jax.jit
def _emb(idx, wflat):
    mesh = plsc.VectorSubcoreMesh(core_axis_name="c", subcore_axis_name="s")
    run = functools.partial(
        pl.kernel,
        mesh=mesh,
        out_type=jax.ShapeDtypeStruct((_B, _S, _E), jnp.float32),
        scratch_types=[
            pltpu.VMEM((32,), jnp.float32),
            pltpu.VMEM((_IDX_BLK, _S), jnp.int32),
            pltpu.VMEM((_OUT_BLK, _S, _E), jnp.float32),
        ],
        compiler_params=pltpu.CompilerParams(needs_layout_passes=False),
    )(_emb_kernel)
    return run(idx, wflat)


def kernel(arg, weight):
    wflat = jnp.pad(weight.reshape(_E * 10), (0, 2))
    return _emb(arg.astype(jnp.int32), wflat)


# speedup vs baseline: 119.9802x; 15.4246x over previous
import functools

import jax
import jax.numpy as jnp
from jax import lax
from jax.experimental import pallas as pl
from jax.experimental.pallas import tpu as pltpu
from jax.experimental.pallas import tpu_sc as plsc

_B = 16384
_S = 200
_E = 3
_NW = 32
_IW = _B // _NW
_JC = 40
_NBLK = _S // _JC


def _emb_kernel(idx_hbm, w_hbm, out_hbm, w_v, idx_v, out_v):
    wid = lax.axis_index("s") * 2 + lax.axis_index("c")
    pltpu.sync_copy(w_hbm, w_v)
    i0 = wid * _IW

    def block_body(b, carry):
        jb = b * _JC
        pltpu.sync_copy(idx_hbm.at[pl.ds(jb, _JC), pl.ds(i0, _IW)], idx_v)

        def j_body(j, c2):
            def k_body(k, c3):
                base = k * 64
                for u in range(4):
                    o = base + u * 16
                    v = idx_v[j, pl.ds(o, 16)]
                    w0 = plsc.load_gather(w_v, [v])
                    w1 = plsc.load_gather(w_v, [v + 16])
                    w2 = plsc.load_gather(w_v, [v + 32])
                    out_v[0, j, pl.ds(o, 16)] = w0
                    out_v[1, j, pl.ds(o, 16)] = w1
                    out_v[2, j, pl.ds(o, 16)] = w2
                return c3

            lax.fori_loop(0, _IW // 64, k_body, 0)
            return c2

        lax.fori_loop(0, _JC, j_body, 0)
        pltpu.sync_copy(
            out_v, out_hbm.at[:, pl.ds(jb, _JC), pl.ds(i0, _IW)]
        )
        return carry

    lax.fori_loop(0, _NBLK, block_body, 0)


@jax.jit
def _emb(idx_t, wcols):
    mesh = plsc.VectorSubcoreMesh(core_axis_name="c", subcore_axis_name="s")
    run = functools.partial(
        pl.kernel,
        mesh=mesh,
        out_type=jax.ShapeDtypeStruct((_E, _S, _B), jnp.float32),
        scratch_types=[
            pltpu.VMEM((3 * 16,), jnp.float32),
            pltpu.VMEM((_JC, _IW), jnp.int32),
            pltpu.VMEM((_E, _JC, _IW), jnp.float32),
        ],
        compiler_params=pltpu.CompilerParams(needs_layout_passes=False),
    )(_emb_kernel)
    return run(idx_t, wcols)


def kernel(arg, weight):
    wcols = jnp.pad(weight.T, ((0, 0), (0, 6))).reshape(3 * 16)
    out_t = _emb(arg.T.astype(jnp.int32), wcols)
    return jnp.transpose(out_t, (2, 1, 0))


# speedup vs baseline: 124.6633x; 1.0390x over previous
import functools

import jax
import jax.numpy as jnp
from jax import lax
from jax.experimental import pallas as pl
from jax.experimental.pallas import tpu as pltpu
from jax.experimental.pallas import tpu_sc as plsc

_B = 16384
_S = 200
_E = 3
_NW = 32
_IW = _B // _NW
_JC = 16
_NMAIN = 12
_JT = _S - _NMAIN * _JC


def _lookup_rows(idx_v, w_v, out_v, n_rows):
    def j_body(j, c2):
        def k_body(k, c3):
            base = k * 128
            for u in range(8):
                o = base + u * 16
                v = idx_v[j, pl.ds(o, 16)]
                w0 = plsc.load_gather(w_v, [v])
                w1 = plsc.load_gather(w_v, [v + 16])
                w2 = plsc.load_gather(w_v, [v + 32])
                out_v[0, j, pl.ds(o, 16)] = w0
                out_v[1, j, pl.ds(o, 16)] = w1
                out_v[2, j, pl.ds(o, 16)] = w2
            return c3

        lax.fori_loop(0, _IW // 128, k_body, 0)
        return c2

    lax.fori_loop(0, n_rows, j_body, 0)


def _emb_kernel(
    idx_hbm, w_hbm, out_hbm, w_v, idx_v, out_v0, out_v1, sem
):
    wid = lax.axis_index("s") * 2 + lax.axis_index("c")
    pltpu.sync_copy(w_hbm, w_v)
    i0 = wid * _IW
    out_bufs = (out_v0, out_v1)

    def t_body(t, carry):
        for p in range(2):
            b = 2 * t + p
            jb = b * _JC
            buf = out_bufs[p]
            pltpu.sync_copy(
                idx_hbm.at[pl.ds(jb, _JC), pl.ds(i0, _IW)], idx_v
            )
            _lookup_rows(idx_v, w_v, buf, _JC)
            if p == 0:
                @pl.when(t > 0)
                def _():
                    pltpu.make_async_copy(
                        out_v1,
                        out_hbm.at[:, pl.ds(0, _JC), pl.ds(i0, _IW)],
                        sem,
                    ).wait()
            else:
                pltpu.make_async_copy(
                    out_v0,
                    out_hbm.at[:, pl.ds(0, _JC), pl.ds(i0, _IW)],
                    sem,
                ).wait()
            pltpu.async_copy(
                buf, out_hbm.at[:, pl.ds(jb, _JC), pl.ds(i0, _IW)], sem
            )
        return carry

    lax.fori_loop(0, _NMAIN // 2, t_body, 0)
    pltpu.make_async_copy(
        out_v1, out_hbm.at[:, pl.ds(0, _JC), pl.ds(i0, _IW)], sem
    ).wait()

    jb = _NMAIN * _JC
    pltpu.sync_copy(
        idx_hbm.at[pl.ds(jb, _JT), pl.ds(i0, _IW)],
        idx_v.at[pl.ds(0, _JT), :],
    )
    _lookup_rows(idx_v, w_v, out_v0, _JT)
    pltpu.sync_copy(
        out_v0.at[:, pl.ds(0, _JT), :],
        out_hbm.at[:, pl.ds(jb, _JT), pl.ds(i0, _IW)],
    )


@jax.jit
def _emb(idx_t, wcols):
    mesh = plsc.VectorSubcoreMesh(core_axis_name="c", subcore_axis_name="s")
    run = functools.partial(
        pl.kernel,
        mesh=mesh,
        out_type=jax.ShapeDtypeStruct((_E, _S, _B), jnp.float32),
        scratch_types=[
            pltpu.VMEM((3 * 16,), jnp.float32),
            pltpu.VMEM((_JC, _IW), jnp.int32),
            pltpu.VMEM((_E, _JC, _IW), jnp.float32),
            pltpu.VMEM((_E, _JC, _IW), jnp.float32),
            pltpu.SemaphoreType.DMA,
        ],
        compiler_params=pltpu.CompilerParams(needs_layout_passes=False),
    )(_emb_kernel)
    return run(idx_t, wcols)


def kernel(arg, weight):
    wcols = jnp.pad(weight.T, ((0, 0), (0, 6))).reshape(3 * 16)
    out_t = _emb(arg.T.astype(jnp.int32), wcols)
    return jnp.transpose(out_t, (2, 1, 0))


# speedup vs baseline: 129.0425x; 1.0351x over previous
import functools

import jax
import jax.numpy as jnp
from jax import lax
from jax.experimental import pallas as pl
from jax.experimental.pallas import tpu as pltpu
from jax.experimental.pallas import tpu_sc as plsc

_B = 16384
_S = 200
_E = 3
_NW = 32
_IW = _B // _NW
_JC = 16
_NMAIN = 12
_JT = _S - _NMAIN * _JC


def _lookup_rows(idx_v, w_v, out_v, n_rows):
    def j_body(j, c2):
        def k_body(k, c3):
            base = k * 128
            for u in range(8):
                o = base + u * 16
                v = idx_v[j, pl.ds(o, 16)]
                w0 = plsc.load_gather(w_v.at[0], [v])
                w1 = plsc.load_gather(w_v.at[1], [v])
                w2 = plsc.load_gather(w_v.at[2], [v])
                out_v[0, j, pl.ds(o, 16)] = w0
                out_v[1, j, pl.ds(o, 16)] = w1
                out_v[2, j, pl.ds(o, 16)] = w2
            return c3

        lax.fori_loop(0, _IW // 128, k_body, 0)
        return c2

    lax.fori_loop(0, n_rows, j_body, 0)


def _emb_kernel(
    idx_hbm, w_hbm, out_hbm, w_v, idx_v, out_v0, out_v1, sem
):
    wid = lax.axis_index("s") * 2 + lax.axis_index("c")
    pltpu.sync_copy(w_hbm, w_v)
    i0 = wid * _IW
    out_bufs = (out_v0, out_v1)

    def t_body(t, carry):
        for p in range(2):
            b = 2 * t + p
            jb = b * _JC
            buf = out_bufs[p]
            pltpu.sync_copy(
                idx_hbm.at[pl.ds(jb, _JC), pl.ds(i0, _IW)], idx_v
            )
            _lookup_rows(idx_v, w_v, buf, _JC)
            if p == 0:
                @pl.when(t > 0)
                def _():
                    pltpu.make_async_copy(
                        out_v1,
                        out_hbm.at[:, pl.ds(0, _JC), pl.ds(i0, _IW)],
                        sem,
                    ).wait()
            else:
                pltpu.make_async_copy(
                    out_v0,
                    out_hbm.at[:, pl.ds(0, _JC), pl.ds(i0, _IW)],
                    sem,
                ).wait()
            pltpu.async_copy(
                buf, out_hbm.at[:, pl.ds(jb, _JC), pl.ds(i0, _IW)], sem
            )
        return carry

    lax.fori_loop(0, _NMAIN // 2, t_body, 0)
    pltpu.make_async_copy(
        out_v1, out_hbm.at[:, pl.ds(0, _JC), pl.ds(i0, _IW)], sem
    ).wait()

    jb = _NMAIN * _JC
    pltpu.sync_copy(
        idx_hbm.at[pl.ds(jb, _JT), pl.ds(i0, _IW)],
        idx_v.at[pl.ds(0, _JT), :],
    )
    _lookup_rows(idx_v, w_v, out_v0, _JT)
    pltpu.sync_copy(
        out_v0.at[:, pl.ds(0, _JT), :],
        out_hbm.at[:, pl.ds(jb, _JT), pl.ds(i0, _IW)],
    )


@jax.jit
def _emb(idx_t, wcols):
    mesh = plsc.VectorSubcoreMesh(core_axis_name="c", subcore_axis_name="s")
    run = functools.partial(
        pl.kernel,
        mesh=mesh,
        out_type=jax.ShapeDtypeStruct((_E, _S, _B), jnp.float32),
        scratch_types=[
            pltpu.VMEM((3, 16), jnp.float32),
            pltpu.VMEM((_JC, _IW), jnp.int32),
            pltpu.VMEM((_E, _JC, _IW), jnp.float32),
            pltpu.VMEM((_E, _JC, _IW), jnp.float32),
            pltpu.SemaphoreType.DMA,
        ],
        compiler_params=pltpu.CompilerParams(needs_layout_passes=False),
    )(_emb_kernel)
    return run(idx_t, wcols)


def kernel(arg, weight):
    wcols = jnp.pad(weight.T, ((0, 0), (0, 6)))
    out_t = _emb(arg.T.astype(jnp.int32), wcols)
    return jnp.transpose(out_t, (2, 1, 0))


# speedup vs baseline: 148.4610x; 1.1505x over previous
import functools

import jax
import jax.numpy as jnp
from jax import lax
from jax.experimental import pallas as pl
from jax.experimental.pallas import tpu as pltpu
from jax.experimental.pallas import tpu_sc as plsc

_B = 16384
_S = 200
_E = 3
_NW = 32
_IW = _B // _NW
_JC = 8
_NB = _S // _JC


def _lookup_rows(idx_v, w_v, out_v, n_rows):
    def j_body(j, c2):
        def k_body(k, c3):
            base = k * 128
            for u in range(8):
                o = base + u * 16
                v = idx_v[j, pl.ds(o, 16)]
                w0 = plsc.load_gather(w_v.at[0], [v])
                w1 = plsc.load_gather(w_v.at[1], [v])
                w2 = plsc.load_gather(w_v.at[2], [v])
                out_v[0, j, pl.ds(o, 16)] = w0
                out_v[1, j, pl.ds(o, 16)] = w1
                out_v[2, j, pl.ds(o, 16)] = w2
            return c3

        lax.fori_loop(0, _IW // 128, k_body, 0)
        return c2

    lax.fori_loop(0, n_rows, j_body, 0)


def _emb_kernel(
    idx_hbm, w_hbm, out_hbm,
    w_v, idx_v0, idx_v1, out_v0, out_v1, si0, si1, so0, so1
):
    wid = lax.axis_index("s") * 2 + lax.axis_index("c")
    pltpu.sync_copy(w_hbm, w_v)
    i0 = wid * _IW
    idx_bufs = (idx_v0, idx_v1)
    idx_sems = (si0, si1)
    out_bufs = (out_v0, out_v1)
    out_sems = (so0, so1)

    def idx_src(b):
        return idx_hbm.at[pl.ds(b * _JC, _JC), pl.ds(i0, _IW)]

    def out_dst(b):
        return out_hbm.at[:, pl.ds(b * _JC, _JC), pl.ds(i0, _IW)]

    pltpu.async_copy(idx_src(0), idx_bufs[0], idx_sems[0])
    for b in range(_NB):
        p = b % 2
        ib, isem = idx_bufs[p], idx_sems[p]
        ob, osem = out_bufs[p], out_sems[p]
        pltpu.make_async_copy(idx_src(b), ib, isem).wait()
        if b + 1 < _NB:
            pltpu.async_copy(
                idx_src(b + 1), idx_bufs[1 - p], idx_sems[1 - p]
            )
        if b >= 2:
            pltpu.make_async_copy(ob, out_dst(b - 2), osem).wait()
        _lookup_rows(ib, w_v, ob, _JC)
        pltpu.async_copy(ob, out_dst(b), osem)
    pltpu.make_async_copy(
        out_bufs[1], out_dst(_NB - 2), out_sems[1]
    ).wait()
    pltpu.make_async_copy(
        out_bufs[0], out_dst(_NB - 1), out_sems[0]
    ).wait()


@jax.jit
def _emb(idx_t, wcols):
    mesh = plsc.VectorSubcoreMesh(core_axis_name="c", subcore_axis_name="s")
    run = functools.partial(
        pl.kernel,
        mesh=mesh,
        out_type=jax.ShapeDtypeStruct((_E, _S, _B), jnp.float32),
        scratch_types=[
            pltpu.VMEM((3, 16), jnp.float32),
            pltpu.VMEM((_JC, _IW), jnp.int32),
            pltpu.VMEM((_JC, _IW), jnp.int32),
            pltpu.VMEM((_E, _JC, _IW), jnp.float32),
            pltpu.VMEM((_E, _JC, _IW), jnp.float32),
            pltpu.SemaphoreType.DMA,
            pltpu.SemaphoreType.DMA,
            pltpu.SemaphoreType.DMA,
            pltpu.SemaphoreType.DMA,
        ],
        compiler_params=pltpu.CompilerParams(needs_layout_passes=False),
    )(_emb_kernel)
    return run(idx_t, wcols)


def kernel(arg, weight):
    wcols = jnp.pad(weight.T, ((0, 0), (0, 6)))
    out_t = _emb(arg.T.astype(jnp.int32), wcols)
    return jnp.transpose(out_t, (2, 1, 0))


# speedup vs baseline: 227.1906x; 1.5303x over previous
import functools

import jax
import jax.numpy as jnp
from jax import lax
from jax.experimental import pallas as pl
from jax.experimental.pallas import tpu as pltpu
from jax.experimental.pallas import tpu_sc as plsc

_B = 16384
_S = 200
_E = 3
_NW = 32
_IW = _B // _NW
_JC = 8
_NB = _S // _JC


def _lookup_rows(idx_v, w_v, out_v, n_rows):
    def j_body(j, c2):
        for u in range(_IW // 16):
            o = u * 16
            v = idx_v[j, pl.ds(o, 16)]
            w0 = plsc.load_gather(w_v.at[0], [v])
            w1 = plsc.load_gather(w_v.at[1], [v])
            w2 = plsc.load_gather(w_v.at[2], [v])
            out_v[0, j, pl.ds(o, 16)] = w0
            out_v[1, j, pl.ds(o, 16)] = w1
            out_v[2, j, pl.ds(o, 16)] = w2
        return c2

    lax.fori_loop(0, n_rows, j_body, 0)


def _emb_kernel(
    idx_hbm, w_hbm, out_hbm,
    w_v, idx_v0, idx_v1, out_v0, out_v1, si0, si1, so0, so1
):
    wid = lax.axis_index("s") * 2 + lax.axis_index("c")
    pltpu.sync_copy(w_hbm, w_v)
    i0 = wid * _IW
    idx_bufs = (idx_v0, idx_v1)
    idx_sems = (si0, si1)
    out_bufs = (out_v0, out_v1)
    out_sems = (so0, so1)

    def idx_src(b):
        return idx_hbm.at[pl.ds(b * _JC, _JC), pl.ds(i0, _IW)]

    def out_dst(b):
        return out_hbm.at[:, pl.ds(b * _JC, _JC), pl.ds(i0, _IW)]

    def block(b, p):
        ib, isem = idx_bufs[p], idx_sems[p]
        ob, osem = out_bufs[p], out_sems[p]
        pltpu.make_async_copy(idx_src(b), ib, isem).wait()

        @pl.when(b + 1 < _NB)
        def _():
            pltpu.async_copy(
                idx_src(b + 1), idx_bufs[1 - p], idx_sems[1 - p]
            )

        @pl.when(b >= 2)
        def _():
            pltpu.make_async_copy(ob, out_dst(b - 2), osem).wait()

        _lookup_rows(ib, w_v, ob, _JC)
        pltpu.async_copy(ob, out_dst(b), osem)

    pltpu.async_copy(idx_src(0), idx_bufs[0], idx_sems[0])

    def b_body(b, carry):
        @pl.when(b % 2 == 0)
        def _():
            block(b, 0)

        @pl.when(b % 2 == 1)
        def _():
            block(b, 1)

        return carry

    lax.fori_loop(0, _NB, b_body, 0)
    pltpu.make_async_copy(
        out_bufs[1], out_dst(_NB - 2), out_sems[1]
    ).wait()
    pltpu.make_async_copy(
        out_bufs[0], out_dst(_NB - 1), out_sems[0]
    ).wait()


@jax.jit
def _emb(idx_t, wcols):
    mesh = plsc.VectorSubcoreMesh(core_axis_name="c", subcore_axis_name="s")
    run = functools.partial(
        pl.kernel,
        mesh=mesh,
        out_type=jax.ShapeDtypeStruct((_E, _S, _B), jnp.float32),
        scratch_types=[
            pltpu.VMEM((3, 16), jnp.float32),
            pltpu.VMEM((_JC, _IW), jnp.int32),
            pltpu.VMEM((_JC, _IW), jnp.int32),
            pltpu.VMEM((_E, _JC, _IW), jnp.float32),
            pltpu.VMEM((_E, _JC, _IW), jnp.float32),
            pltpu.SemaphoreType.DMA,
            pltpu.SemaphoreType.DMA,
            pltpu.SemaphoreType.DMA,
            pltpu.SemaphoreType.DMA,
        ],
        compiler_params=pltpu.CompilerParams(needs_layout_passes=False),
    )(_emb_kernel)
    return run(idx_t, wcols)


def kernel(arg, weight):
    wcols = jnp.pad(weight.T, ((0, 0), (0, 6)))
    out_t = _emb(arg.T.astype(jnp.int32), wcols)
    return jnp.transpose(out_t, (2, 1, 0))
